# Initial kernel scaffold; baseline (speedup 1.0000x reference)
import jax
import jax.numpy as jnp
from jax.experimental import pallas as pl


def kernel(points, time_indice, est_poses, gt_poses):
    raise NotImplementedError("write your pallas kernel here")


# speedup vs baseline: 886.1083x; 886.1083x over previous
import jax
import jax.numpy as jnp
from jax.experimental import pallas as pl
from jax.experimental.pallas import tpu as pltpu

_EPS = 1e-7
_N = 8192
_F = 10
_BM = 256
_NT = _N // _BM
_PREC = jax.lax.Precision.HIGHEST


def _fused_kernel(ph_ref, phT_ref, ti_col_ref, ti_row_ref,
                  gt_flat_ref, estT_flatT_ref, gtT_flatT_ref,
                  cham_ref, l2_ref, g_scr, gn2_scr):
    ph = ph_ref[...]
    phT = phT_ref[...]
    ti_col = ti_col_ref[...]
    ti_row = ti_row_ref[...]
    gt_flat = gt_flat_ref[...]
    estT_flatT = estT_flatT_ref[...]
    gtT_flatT = gtT_flatT_ref[...]

    iota_col = jax.lax.broadcasted_iota(jnp.int32, (_N, _F), 1)
    onehot_col = (iota_col == ti_col).astype(jnp.float32)
    iota_row = jax.lax.broadcasted_iota(jnp.int32, (_F, _N), 0)
    onehot_rowT = (iota_row == ti_row).astype(jnp.float32)

    BeT = jnp.dot(estT_flatT, onehot_rowT,
                  preferred_element_type=jnp.float32, precision=_PREC)
    BgT = jnp.dot(gtT_flatT, onehot_rowT,
                  preferred_element_type=jnp.float32, precision=_PREC)
    Bg = jnp.dot(onehot_col, gt_flat,
                 preferred_element_type=jnp.float32, precision=_PREC)

    def rows_from(BT):
        rs = []
        for k in range(3):
            acc = BT[4 * k:4 * k + 1, :] * phT[0:1, :]
            for j in range(1, 4):
                acc = acc + BT[4 * k + j:4 * k + j + 1, :] * phT[j:j + 1, :]
            rs.append(acc)
        return jnp.concatenate(rs, axis=0)

    estT = rows_from(BeT)
    gtT = rows_from(BgT)
    en2_row = jnp.sum(estT * estT, axis=0, keepdims=True)
    g_scr[...] = -2.0 * gtT

    gn2_col = jnp.zeros((_N, 1), jnp.float32)
    for k in range(3):
        acc = Bg[:, 4 * k:4 * k + 1] * ph[:, 0:1]
        for j in range(1, 4):
            acc = acc + Bg[:, 4 * k + j:4 * k + j + 1] * ph[:, j:j + 1]
        gn2_col = gn2_col + acc * acc
    gn2_scr[...] = gn2_col

    w_row = (ti_row == 1).astype(jnp.float32)
    cnt = jnp.sum(w_row)

    def body(g, carry):
        run2, s1 = carry
        Gt = g_scr[:, pl.ds(g * _BM, _BM)]
        gn2_t = gn2_scr[pl.ds(g * _BM, _BM), :]
        ti_t = ti_col_ref[pl.ds(g * _BM, _BM), :]
        wt = (ti_t == 1).astype(jnp.float32)
        cross = jax.lax.dot_general(
            Gt, estT, (((0,), (0,)), ((), ())),
            preferred_element_type=jnp.float32,
            precision=jax.lax.Precision.DEFAULT)
        d2 = (cross + en2_row) + gn2_t
        d1 = jnp.min(d2, axis=1, keepdims=True)
        s1 = s1 + jnp.sum(wt * d1, keepdims=True).reshape(1, 1)
        run2 = jnp.minimum(run2, jnp.min(d2, axis=0, keepdims=True))
        return run2, s1

    init = (jnp.full((1, _N), jnp.inf, jnp.float32),
            jnp.zeros((1, 1), jnp.float32))
    run2, s1 = jax.lax.fori_loop(0, _NT, body, init)
    s2 = jnp.sum(w_row * run2, keepdims=True).reshape(1, 1)

    denom = cnt + _EPS
    cham_ref[0, 0] = ((s1[0, 0] + s2[0, 0]) / denom) * 0.5

    diffT = estT - gtT
    nrm = jnp.sqrt(jnp.sum(diffT * diffT, axis=0, keepdims=True))
    l2_ref[0, 0] = jnp.sum(w_row * nrm) / denom


def kernel(points, time_indice, est_poses, gt_poses):
    ph = jnp.concatenate(
        [points, jnp.ones((_N, 1), points.dtype)], axis=1)
    phT = ph.T
    ti_col = time_indice.reshape(_N, 1)
    ti_row = time_indice.reshape(1, _N)
    gt_flat = gt_poses[:, :3, :4].reshape(_F, 12)
    estT_flatT = est_poses[:, :3, :4].reshape(_F, 12).T
    gtT_flatT = gt_flat.T

    cham, l2 = pl.pallas_call(
        _fused_kernel,
        out_shape=(
            jax.ShapeDtypeStruct((1, 1), jnp.float32),
            jax.ShapeDtypeStruct((1, 1), jnp.float32),
        ),
        in_specs=[
            pl.BlockSpec((_N, 4), lambda: (0, 0)),
            pl.BlockSpec((4, _N), lambda: (0, 0)),
            pl.BlockSpec((_N, 1), lambda: (0, 0)),
            pl.BlockSpec((1, _N), lambda: (0, 0)),
            pl.BlockSpec((_F, 12), lambda: (0, 0)),
            pl.BlockSpec((12, _F), lambda: (0, 0)),
            pl.BlockSpec((12, _F), lambda: (0, 0)),
        ],
        out_specs=(
            pl.BlockSpec(memory_space=pltpu.SMEM),
            pl.BlockSpec(memory_space=pltpu.SMEM),
        ),
        scratch_shapes=[
            pltpu.VMEM((3, _N), jnp.float32),
            pltpu.VMEM((_N, 1), jnp.float32),
        ],
    )(ph, phT, ti_col, ti_row, gt_flat, estT_flatT, gtT_flatT)
    return cham[0, 0], l2[0, 0]


# speedup vs baseline: 957.8587x; 1.0810x over previous
import jax
import jax.numpy as jnp
from jax.experimental import pallas as pl
from jax.experimental.pallas import tpu as pltpu

_EPS = 1e-7
_N = 8192
_F = 10
_BM = 512
_NT = _N // _BM
_PREC = jax.lax.Precision.HIGHEST


def _fused_kernel(ph_ref, phT_ref, ti_col_ref, ti_row_ref,
                  gt_flat_ref, estT_flatT_ref, gtT_flatT_ref,
                  cham_ref, l2_ref, g_scr, gn2_scr):
    ph = ph_ref[...]
    phT = phT_ref[...]
    ti_col = ti_col_ref[...]
    ti_row = ti_row_ref[...]
    gt_flat = gt_flat_ref[...]
    estT_flatT = estT_flatT_ref[...]
    gtT_flatT = gtT_flatT_ref[...]

    iota_col = jax.lax.broadcasted_iota(jnp.int32, (_N, _F), 1)
    onehot_col = (iota_col == ti_col).astype(jnp.float32)
    iota_row = jax.lax.broadcasted_iota(jnp.int32, (_F, _N), 0)
    onehot_rowT = (iota_row == ti_row).astype(jnp.float32)

    BeT = jnp.dot(estT_flatT, onehot_rowT,
                  preferred_element_type=jnp.float32, precision=_PREC)
    BgT = jnp.dot(gtT_flatT, onehot_rowT,
                  preferred_element_type=jnp.float32, precision=_PREC)
    Bg = jnp.dot(onehot_col, gt_flat,
                 preferred_element_type=jnp.float32, precision=_PREC)

    def rows_from(BT):
        rs = []
        for k in range(3):
            acc = BT[4 * k:4 * k + 1, :] * phT[0:1, :]
            for j in range(1, 4):
                acc = acc + BT[4 * k + j:4 * k + j + 1, :] * phT[j:j + 1, :]
            rs.append(acc)
        return jnp.concatenate(rs, axis=0)

    estT = rows_from(BeT)
    gtT = rows_from(BgT)
    en2_row = jnp.sum(estT * estT, axis=0, keepdims=True)
    g_scr[...] = -2.0 * gtT

    gn2_col = jnp.zeros((_N, 1), jnp.float32)
    for k in range(3):
        acc = Bg[:, 4 * k:4 * k + 1] * ph[:, 0:1]
        for j in range(1, 4):
            acc = acc + Bg[:, 4 * k + j:4 * k + j + 1] * ph[:, j:j + 1]
        gn2_col = gn2_col + acc * acc
    gn2_scr[...] = gn2_col

    w_row = (ti_row == 1).astype(jnp.float32)
    cnt = jnp.sum(w_row)

    def body(g, carry):
        run2, s1 = carry
        Gt = g_scr[:, pl.ds(g * _BM, _BM)]
        gn2_t = gn2_scr[pl.ds(g * _BM, _BM), :]
        ti_t = ti_col_ref[pl.ds(g * _BM, _BM), :]
        wt = (ti_t == 1).astype(jnp.float32)
        cross = jax.lax.dot_general(
            Gt, estT, (((0,), (0,)), ((), ())),
            preferred_element_type=jnp.float32,
            precision=jax.lax.Precision.DEFAULT)
        d2 = (cross + en2_row) + gn2_t
        d1 = jnp.min(d2, axis=1, keepdims=True)
        s1 = s1 + jnp.sum(wt * d1, keepdims=True).reshape(1, 1)
        run2 = jnp.minimum(run2, jnp.min(d2, axis=0, keepdims=True))
        return run2, s1

    init = (jnp.full((1, _N), jnp.inf, jnp.float32),
            jnp.zeros((1, 1), jnp.float32))
    run2, s1 = jax.lax.fori_loop(0, _NT, body, init)
    s2 = jnp.sum(w_row * run2, keepdims=True).reshape(1, 1)

    denom = cnt + _EPS
    cham_ref[0, 0] = ((s1[0, 0] + s2[0, 0]) / denom) * 0.5

    diffT = estT - gtT
    nrm = jnp.sqrt(jnp.sum(diffT * diffT, axis=0, keepdims=True))
    l2_ref[0, 0] = jnp.sum(w_row * nrm) / denom


def kernel(points, time_indice, est_poses, gt_poses):
    ph = jnp.concatenate(
        [points, jnp.ones((_N, 1), points.dtype)], axis=1)
    phT = ph.T
    ti_col = time_indice.reshape(_N, 1)
    ti_row = time_indice.reshape(1, _N)
    gt_flat = gt_poses[:, :3, :4].reshape(_F, 12)
    estT_flatT = est_poses[:, :3, :4].reshape(_F, 12).T
    gtT_flatT = gt_flat.T

    cham, l2 = pl.pallas_call(
        _fused_kernel,
        out_shape=(
            jax.ShapeDtypeStruct((1, 1), jnp.float32),
            jax.ShapeDtypeStruct((1, 1), jnp.float32),
        ),
        in_specs=[
            pl.BlockSpec((_N, 4), lambda: (0, 0)),
            pl.BlockSpec((4, _N), lambda: (0, 0)),
            pl.BlockSpec((_N, 1), lambda: (0, 0)),
            pl.BlockSpec((1, _N), lambda: (0, 0)),
            pl.BlockSpec((_F, 12), lambda: (0, 0)),
            pl.BlockSpec((12, _F), lambda: (0, 0)),
            pl.BlockSpec((12, _F), lambda: (0, 0)),
        ],
        out_specs=(
            pl.BlockSpec(memory_space=pltpu.SMEM),
            pl.BlockSpec(memory_space=pltpu.SMEM),
        ),
        scratch_shapes=[
            pltpu.VMEM((3, _N), jnp.float32),
            pltpu.VMEM((_N, 1), jnp.float32),
        ],
    )(ph, phT, ti_col, ti_row, gt_flat, estT_flatT, gtT_flatT)
    return cham[0, 0], l2[0, 0]
